# Initial kernel scaffold; baseline (speedup 1.0000x reference)
import jax
import jax.numpy as jnp
from jax.experimental import pallas as pl


def kernel(x, upsample_index):
    raise NotImplementedError("write your pallas kernel here")


# speedup vs baseline: 2.5030x; 2.5030x over previous
import jax
import jax.numpy as jnp
from jax import lax
from jax.experimental import pallas as pl
from jax.experimental.pallas import tpu as pltpu
from jax.experimental.pallas import tpu_sc as plsc

_N_COARSE = 40962
_N_NEW = 122880
_D = 128
_LANES = 16
_NC, _NS = 2, 16
_NW = _NC * _NS

_ROWS_W = _N_NEW // _NW
_B = 256
_NCH = _ROWS_W // _B
_CPY_W = _N_COARSE // _NW
_CB = 256
_NCPY = _CPY_W // _CB
_CPY_REM = _N_COARSE - _CPY_W * _NW


def _body(x, iflat, out, gbuf, obuf, idxv, sem):
    cid = lax.axis_index("c")
    sid = lax.axis_index("s")
    wid = sid * _NC + cid

    def cpy(t, carry):
        base = wid * _CPY_W + t * _CB
        pltpu.sync_copy(x.at[pl.ds(base, _CB)], obuf)
        pltpu.sync_copy(obuf, out.at[pl.ds(base, _CB)])
        return carry

    lax.fori_loop(0, _NCPY, cpy, 0)

    @pl.when(wid == _NW - 1)
    def _rem():
        pltpu.sync_copy(x.at[pl.ds(_NW * _CPY_W, _CPY_REM)],
                        obuf.at[pl.ds(0, _CPY_REM)])
        pltpu.sync_copy(obuf.at[pl.ds(0, _CPY_REM)],
                        out.at[pl.ds(_NW * _CPY_W, _CPY_REM)])

    def chunk(t, carry):
        base = wid * _ROWS_W + t * _B
        pltpu.sync_copy(iflat.at[pl.ds(2 * base, 2 * _B)], idxv)
        pltpu.async_copy(x.at[idxv], gbuf, sem).wait()

        def avg(r, c2):
            for v in range(_D // _LANES):
                sl = pl.ds(v * _LANES, _LANES)
                a = gbuf[2 * r, sl]
                b = gbuf[2 * r + 1, sl]
                obuf[r, sl] = (a + b) * 0.5
            return c2

        lax.fori_loop(0, _B, avg, 0)
        pltpu.sync_copy(obuf, out.at[pl.ds(_N_COARSE + base, _B)])
        return carry

    lax.fori_loop(0, _NCH, chunk, 0)


@jax.jit
def kernel(x, upsample_index):
    iflat = upsample_index.reshape(-1)
    f = pl.kernel(
        _body,
        out_type=jax.ShapeDtypeStruct((_N_COARSE + _N_NEW, _D), jnp.float32),
        mesh=plsc.VectorSubcoreMesh(
            core_axis_name="c", subcore_axis_name="s",
            num_cores=_NC, num_subcores=_NS,
        ),
        scratch_types=[
            pltpu.VMEM((2 * _B, _D), jnp.float32),
            pltpu.VMEM((_B, _D), jnp.float32),
            pltpu.VMEM((2 * _B,), jnp.int32),
            pltpu.SemaphoreType.DMA,
        ],
        compiler_params=pltpu.CompilerParams(use_tc_tiling_on_sc=False),
    )
    return f(x, iflat)


# speedup vs baseline: 3.1003x; 1.2386x over previous
import jax
import jax.numpy as jnp
from jax import lax
from jax.experimental import pallas as pl
from jax.experimental.pallas import tpu as pltpu
from jax.experimental.pallas import tpu_sc as plsc

_N_COARSE = 40962
_N_NEW = 122880
_D = 128
_LANES = 16
_NC, _NS = 2, 16
_NW = _NC * _NS

_ROWS_W = _N_NEW // _NW
_B = 128
_NCH = _ROWS_W // _B
_NPAIR = _NCH // 2
_CPY_W = _N_COARSE // _NW
_CB = 128
_NCPY = _CPY_W // _CB
_CPY_REM = _N_COARSE - _CPY_W * _NW


def _body(x, idx3, out, gbuf0, gbuf1, obuf0, obuf1, cbuf, idxall,
          semg0, semg1, sems0, sems1):
    gbufs = (gbuf0, gbuf1)
    obufs = (obuf0, obuf1)
    semg = (semg0, semg1)
    sems = (sems0, sems1)
    cid = lax.axis_index("c")
    sid = lax.axis_index("s")
    wid = sid * _NC + cid

    pltpu.sync_copy(idx3.at[wid], idxall)

    def start_gather(c, i):
        pltpu.async_copy(x.at[idxall.at[c]], gbufs[i], semg[i])

    def wait_gather(i):
        pltpu.make_async_copy(x.at[pl.ds(0, 2 * _B)], gbufs[i], semg[i]).wait()

    def start_store(c, i):
        base = wid * _ROWS_W + c * _B
        pltpu.async_copy(obufs[i], out.at[pl.ds(_N_COARSE + base, _B)], sems[i])

    def wait_store(i):
        pltpu.make_async_copy(obufs[i], out.at[pl.ds(_N_COARSE, _B)],
                              sems[i]).wait()

    def avg(i):
        g = gbufs[i]
        o = obufs[i]

        def rows(r, cc):
            for rr in range(2):
                row = 2 * r + rr
                for v in range(_D // _LANES):
                    sl = pl.ds(v * _LANES, _LANES)
                    o[row, sl] = (g[2 * row, sl] + g[2 * row + 1, sl]) * 0.5
            return cc

        lax.fori_loop(0, _B // 2, rows, 0)

    start_gather(0, 0)
    start_gather(1, 1)

    def cpy(t, carry):
        base = wid * _CPY_W + t * _CB
        pltpu.sync_copy(x.at[pl.ds(base, _CB)], cbuf)
        pltpu.sync_copy(cbuf, out.at[pl.ds(base, _CB)])
        return carry

    lax.fori_loop(0, _NCPY, cpy, 0)

    @pl.when(wid == _NW - 1)
    def _rem():
        pltpu.sync_copy(x.at[pl.ds(_NW * _CPY_W, _CPY_REM)],
                        cbuf.at[pl.ds(0, _CPY_REM)])
        pltpu.sync_copy(cbuf.at[pl.ds(0, _CPY_REM)],
                        out.at[pl.ds(_NW * _CPY_W, _CPY_REM)])

    def pair(p, carry):
        for i in range(2):
            c = 2 * p + i
            wait_gather(i)

            @pl.when(c >= 2)
            def _ws():
                wait_store(i)

            avg(i)
            start_store(c, i)

            @pl.when(p < _NPAIR - 1)
            def _ng():
                start_gather(c + 2, i)

        return carry

    lax.fori_loop(0, _NPAIR, pair, 0)
    wait_store(0)
    wait_store(1)


@jax.jit
def kernel(x, upsample_index):
    idx3 = upsample_index.reshape(_NW, _NCH, 2 * _B)
    f = pl.kernel(
        _body,
        out_type=jax.ShapeDtypeStruct((_N_COARSE + _N_NEW, _D), jnp.float32),
        mesh=plsc.VectorSubcoreMesh(
            core_axis_name="c", subcore_axis_name="s",
            num_cores=_NC, num_subcores=_NS,
        ),
        scratch_types=[
            pltpu.VMEM((2 * _B, _D), jnp.float32),
            pltpu.VMEM((2 * _B, _D), jnp.float32),
            pltpu.VMEM((_B, _D), jnp.float32),
            pltpu.VMEM((_B, _D), jnp.float32),
            pltpu.VMEM((_CB, _D), jnp.float32),
            pltpu.VMEM((_NCH, 2 * _B), jnp.int32),
            pltpu.SemaphoreType.DMA,
            pltpu.SemaphoreType.DMA,
            pltpu.SemaphoreType.DMA,
            pltpu.SemaphoreType.DMA,
        ],
        compiler_params=pltpu.CompilerParams(use_tc_tiling_on_sc=False),
    )
    return f(x, idx3)
